# baseline (speedup 1.0000x reference)
import jax
import jax.numpy as jnp
from jax import lax
from jax.experimental import pallas as pl
from jax.experimental.pallas import tpu as pltpu
from jax.experimental.pallas import tpu_sc as plsc

BATCH = 16384
EMBED_DIM = 32
PACK = 128 // EMBED_DIM
NC = 2
NS = 16
NW = NC * NS
BPW = BATCH // NW
CHUNK = 128
NCH = BPW // CHUNK


def _sc_body(uidx_hbm, pidx_hbm, uemb_hbm, ub_hbm, pemb_hbm, pb_hbm,
             partials_hbm, bsum_hbm,
             uidx_v, pidx_v, usup_v, psup_v,
             urows_v, prows_v, ubv, pbv, bsumv, accv, sems):
    wid = lax.axis_index("c") * NS + lax.axis_index("s")
    base = wid * BPW
    pltpu.sync_copy(uidx_hbm.at[pl.ds(base, BPW)], uidx_v)
    pltpu.sync_copy(pidx_hbm.at[pl.ds(base, BPW)], pidx_v)

    def sup_body(i, carry):
        s = pl.ds(i * 16, 16)
        usup_v[s] = lax.shift_right_logical(uidx_v[s], 2)
        psup_v[s] = lax.shift_right_logical(pidx_v[s], 2)
        return carry
    lax.fori_loop(0, BPW // 16, sup_body, 0)

    def fire(c, buf):
        s = pl.ds(c * CHUNK, CHUNK)
        sem = sems.at[buf]
        return (
            pltpu.async_copy(uemb_hbm.at[usup_v.at[s]], urows_v.at[buf], sem),
            pltpu.async_copy(pemb_hbm.at[psup_v.at[s]], prows_v.at[buf], sem),
            pltpu.async_copy(ub_hbm.at[uidx_v.at[s]], ubv.at[buf], sem),
            pltpu.async_copy(pb_hbm.at[pidx_v.at[s]], pbv.at[buf], sem),
        )

    acc = jnp.zeros((16,), jnp.float32)
    inflight = fire(0, 0)
    for c in range(NCH):
        cur, buf = inflight, c % 2
        if c + 1 < NCH:
            nxt = fire(c + 1, (c + 1) % 2)
        for cp in cur:
            cp.wait()

        def dot_body(i, a, _buf=buf, _c=c):
            uvec = uidx_v[pl.ds(_c * CHUNK + i * 16, 16)]
            pvec = pidx_v[pl.ds(_c * CHUNK + i * 16, 16)]
            for t in range(16):
                cu = (uvec[t] & (PACK - 1)) * EMBED_DIM
                cp_ = (pvec[t] & (PACK - 1)) * EMBED_DIM
                k = i * 16 + t
                a = a + urows_v[_buf, k, pl.ds(cu, 16)] * prows_v[_buf, k, pl.ds(cp_, 16)]
                a = a + urows_v[_buf, k, pl.ds(cu + 16, 16)] * prows_v[_buf, k, pl.ds(cp_ + 16, 16)]
            return a
        acc = lax.fori_loop(0, CHUNK // 16, dot_body, acc)

        def bias_body(i, carry, _buf=buf, _c=c):
            s16 = pl.ds(i * 16, 16)
            bsumv[pl.ds(_c * CHUNK + i * 16, 16)] = ubv[_buf, s16] + pbv[_buf, s16]
            return carry
        lax.fori_loop(0, CHUNK // 16, bias_body, 0)
        if c + 1 < NCH:
            inflight = nxt

    accv[...] = acc
    pltpu.sync_copy(accv, partials_hbm.at[wid])
    pltpu.sync_copy(bsumv, bsum_hbm.at[pl.ds(base, BPW)])


def _sc_stage(u_idx, p_idx, uemb2, ub_flat, pemb2, pb_flat):
    mesh = plsc.VectorSubcoreMesh(core_axis_name="c", subcore_axis_name="s")
    f = pl.kernel(
        _sc_body,
        mesh=mesh,
        out_type=[
            jax.ShapeDtypeStruct((NW, 16), jnp.float32),
            jax.ShapeDtypeStruct((BATCH,), jnp.float32),
        ],
        scratch_types=[
            pltpu.VMEM((BPW,), jnp.int32),
            pltpu.VMEM((BPW,), jnp.int32),
            pltpu.VMEM((BPW,), jnp.int32),
            pltpu.VMEM((BPW,), jnp.int32),
            pltpu.VMEM((2, CHUNK, 128), jnp.float32),
            pltpu.VMEM((2, CHUNK, 128), jnp.float32),
            pltpu.VMEM((2, CHUNK), jnp.float32),
            pltpu.VMEM((2, CHUNK), jnp.float32),
            pltpu.VMEM((BPW,), jnp.float32),
            pltpu.VMEM((16,), jnp.float32),
            pltpu.SemaphoreType.DMA((2,)),
        ],
    )
    return f(u_idx, p_idx, uemb2, ub_flat, pemb2, pb_flat)


def _tc_body(partials_ref, bsum_ref, out_ref):
    s = jnp.sum(partials_ref[...])
    out_ref[...] = jax.nn.sigmoid(bsum_ref[...] + s)


def kernel(inputs, user_embedding, user_bias, place_embedding, place_bias):
    u_idx = inputs[:, 0]
    p_idx = inputs[:, 1]
    uemb2 = user_embedding.reshape(-1, 128)
    pemb2 = place_embedding.reshape(-1, 128)
    partials, bsum = _sc_stage(
        u_idx, p_idx, uemb2, user_bias.reshape(-1), pemb2, place_bias.reshape(-1))
    out = pl.pallas_call(
        _tc_body,
        out_shape=jax.ShapeDtypeStruct((128, 128), jnp.float32),
    )(partials, bsum.reshape(128, 128))
    return out.reshape(BATCH, 1)
